# Initial kernel scaffold; baseline (speedup 1.0000x reference)
import jax
import jax.numpy as jnp
from jax.experimental import pallas as pl


def kernel(x, edge_index, scg_pair, gpt_pair, esm_pair, pair_idx, W1, b1, W2, b2, Wc1, bc1, Wc2, bc2, Wc3, bc3):
    raise NotImplementedError("write your pallas kernel here")


# speedup vs baseline: 11.9197x; 11.9197x over previous
import functools

import jax
import jax.numpy as jnp
from jax import lax
from jax.experimental import pallas as pl
from jax.experimental.pallas import tpu as pltpu
from jax.experimental.pallas import tpu_sc as plsc

NCORE = 2
NSUB = 16
KI = 80


def _sc_mesh():
    return plsc.VectorSubcoreMesh(core_axis_name="c", subcore_axis_name="s")


def _make_deg_kernel(Np, E):
    per_w = E // (NCORE * NSUB)
    nchunk = per_w // KI
    rpt = Np // NSUB

    @functools.partial(
        pl.kernel,
        out_type=jax.ShapeDtypeStruct((NCORE * Np, 128), jnp.float32),
        mesh=_sc_mesh(),
        scratch_types=[
            pltpu.VMEM_SHARED((Np, 128), jnp.float32),
            pltpu.VMEM((KI,), jnp.int32),
            pltpu.VMEM((KI, 128), jnp.float32),
        ],
    )
    def deg_kernel(dst_ref, zeros_ref, ones_ref, out_ref, acc, idx_v, ones_v):
        c = lax.axis_index("c")
        s = lax.axis_index("s")
        pltpu.sync_copy(zeros_ref.at[pl.ds(s * rpt, rpt)], acc.at[pl.ds(s * rpt, rpt)])
        pltpu.sync_copy(ones_ref, ones_v)
        plsc.subcore_barrier()
        wid = c * NSUB + s
        base0 = wid * per_w

        def body(g, carry):
            pltpu.sync_copy(dst_ref.at[pl.ds(base0 + g * KI, KI)], idx_v)
            pltpu.sync_copy(ones_v, acc.at[idx_v], add=True)
            return carry

        lax.fori_loop(0, nchunk, body, 0)
        plsc.subcore_barrier()
        pltpu.sync_copy(acc.at[pl.ds(s * rpt, rpt)],
                        out_ref.at[pl.ds(c * Np + s * rpt, rpt)])

    return deg_kernel


def _make_agg_kernel(Np, E, F):
    per_t = E // NSUB
    nchunk = per_t // KI
    rpt = Np // NSUB

    @functools.partial(
        pl.kernel,
        out_type=jax.ShapeDtypeStruct((NCORE * Np, F), jnp.float32),
        mesh=_sc_mesh(),
        scratch_types=[
            pltpu.VMEM_SHARED((Np, F), jnp.float32),
            pltpu.VMEM((KI,), jnp.int32),
            pltpu.VMEM((KI,), jnp.int32),
            pltpu.VMEM((KI,), jnp.int32),
            pltpu.VMEM((KI,), jnp.int32),
            pltpu.VMEM((KI, F), jnp.float32),
            pltpu.VMEM((KI, F), jnp.float32),
            pltpu.SemaphoreType.DMA,
            pltpu.SemaphoreType.DMA,
        ],
    )
    def agg_kernel(ht_ref, srcall_ref, dst_ref, out_ref, acc, src0, src1,
                   dst0, dst1, rows0, rows1, sem0, sem1):
        c = lax.axis_index("c")
        s = lax.axis_index("s")
        row0 = c * Np
        pltpu.sync_copy(ht_ref.at[pl.ds(row0 + s * rpt, rpt)],
                        acc.at[pl.ds(s * rpt, rpt)])
        plsc.subcore_barrier()
        sbase = c * E + s * per_t
        dbase = s * per_t

        def stage_fire(g, sv, dv, rv, sem):
            pltpu.sync_copy(srcall_ref.at[pl.ds(sbase + g * KI, KI)], sv)
            pltpu.sync_copy(dst_ref.at[pl.ds(dbase + g * KI, KI)], dv)
            pltpu.make_async_copy(ht_ref.at[sv], rv, sem).start()

        stage_fire(0, src0, dst0, rows0, sem0)

        def body(gp, carry):
            g0 = 2 * gp
            stage_fire(g0 + 1, src1, dst1, rows1, sem1)
            pltpu.make_async_copy(ht_ref.at[src0], rows0, sem0).wait()
            pltpu.sync_copy(rows0, acc.at[dst0], add=True)

            @pl.when(g0 + 2 < nchunk)
            def _():
                stage_fire(g0 + 2, src0, dst0, rows0, sem0)

            pltpu.make_async_copy(ht_ref.at[src1], rows1, sem1).wait()
            pltpu.sync_copy(rows1, acc.at[dst1], add=True)
            return carry

        lax.fori_loop(0, nchunk // 2, body, 0)
        plsc.subcore_barrier()
        pltpu.sync_copy(acc.at[pl.ds(s * rpt, rpt)],
                        out_ref.at[pl.ds(row0 + s * rpt, rpt)])

    return agg_kernel


def _make_pair_kernel(Np, P, D):
    per_w = P // (NCORE * NSUB)
    gk = 128
    ng = per_w // gk

    @functools.partial(
        pl.kernel,
        out_type=jax.ShapeDtypeStruct((P, D), jnp.float32),
        mesh=_sc_mesh(),
        scratch_types=[
            pltpu.VMEM((gk,), jnp.int32),
            pltpu.VMEM((gk, D), jnp.float32),
            pltpu.SemaphoreType.DMA,
        ],
    )
    def pair_kernel(h_ref, idx_ref, out_ref, idx_v, rows_v, sem):
        c = lax.axis_index("c")
        s = lax.axis_index("s")
        wid = c * NSUB + s

        def body(g, carry):
            base = wid * per_w + g * gk
            pltpu.sync_copy(idx_ref.at[pl.ds(base, gk)], idx_v)
            pltpu.async_copy(h_ref.at[idx_v], rows_v, sem).wait()
            pltpu.sync_copy(rows_v, out_ref.at[pl.ds(base, gk)])
            return carry

        lax.fori_loop(0, ng, body, 0)

    return pair_kernel


def _h1_body(deg_ref, x_ref, w_ref, o_ref):
    dis = lax.rsqrt(deg_ref[0, :, 0:1] + deg_ref[1, :, 0:1] + 1.0)
    h = jnp.dot(x_ref[...], w_ref[...], preferred_element_type=jnp.float32,
                precision=lax.Precision.HIGHEST)
    ht = h * dis
    F = ht.shape[1] // 2
    o_ref[0] = ht[:, :F]
    o_ref[1] = ht[:, F:]


def _mid_body(deg_ref, s_ref, b_ref, w_ref, o_ref):
    dis = lax.rsqrt(deg_ref[0, :, 0:1] + deg_ref[1, :, 0:1] + 1.0)
    ssum = jnp.concatenate([s_ref[0], s_ref[1]], axis=1)
    x2 = jnp.maximum(dis * ssum + b_ref[...], 0.0)
    h2 = jnp.dot(x2, w_ref[...], preferred_element_type=jnp.float32,
                 precision=lax.Precision.HIGHEST)
    ht2 = h2 * dis
    F = ht2.shape[1] // 2
    o_ref[0] = ht2[:, :F]
    o_ref[1] = ht2[:, F:]


def _fin_body(deg_ref, s_ref, b_ref, o_ref):
    dis = lax.rsqrt(deg_ref[0, :, 0:1] + deg_ref[1, :, 0:1] + 1.0)
    ssum = jnp.concatenate([s_ref[0], s_ref[1]], axis=1)
    o_ref[...] = dis * ssum + b_ref[...]


def _mlp_body(g_ref, s_ref, e_ref, p_ref, wg, ws, we, wp, b1r, w2, b2r, w3,
              b3r, o_ref):
    dot = functools.partial(jnp.dot, preferred_element_type=jnp.float32,
                            precision=lax.Precision.HIGHEST)
    z = (dot(g_ref[...], wg[...]) + dot(s_ref[...], ws[...]) +
         dot(e_ref[...], we[...]) + dot(p_ref[...], wp[...]) + b1r[...])
    z = jnp.maximum(z, 0.0)
    z2 = jnp.maximum(dot(z, w2[...]) + b2r[...], 0.0)
    o_ref[...] = dot(z2, w3[...]) + b3r[...]


def kernel(x, edge_index, scg_pair, gpt_pair, esm_pair, pair_idx, W1, b1, W2,
           b2, Wc1, bc1, Wc2, bc2, Wc3, bc3):
    N, Fin = x.shape
    E = edge_index.shape[1]
    B = pair_idx.shape[0]
    H = W1.shape[1]
    D = W2.shape[1]
    F = D // 2
    Np = ((N + 1023) // 1024) * 1024
    ei = edge_index.astype(jnp.int32)
    src_e = ei[0]
    dst_e = ei[1]
    srcall = jnp.concatenate([src_e, src_e + Np])
    pidx = pair_idx.astype(jnp.int32).reshape(B * 2)
    xp = jnp.pad(x, ((0, Np - N), (0, 0)))

    deg_kernel = _make_deg_kernel(Np, E)
    zeros_n = jnp.zeros((Np, 128), jnp.float32)
    ones_k = jnp.ones((KI, 128), jnp.float32)
    deg_parts = deg_kernel(dst_e, zeros_n, ones_k).reshape(NCORE, Np, 128)

    RB = 1024
    grid = (Np // RB,)
    deg_spec = pl.BlockSpec((NCORE, RB, 128), lambda i: (0, i, 0))
    half_spec = pl.BlockSpec((NCORE, RB, F), lambda i: (0, i, 0))
    ht1 = pl.pallas_call(
        _h1_body,
        grid=grid,
        in_specs=[
            deg_spec,
            pl.BlockSpec((RB, Fin), lambda i: (i, 0)),
            pl.BlockSpec((Fin, H), lambda i: (0, 0)),
        ],
        out_specs=half_spec,
        out_shape=jax.ShapeDtypeStruct((NCORE, Np, F), jnp.float32),
    )(deg_parts, xp, W1)

    agg_kernel = _make_agg_kernel(Np, E, F)
    sum1 = agg_kernel(ht1.reshape(NCORE * Np, F), srcall,
                      dst_e).reshape(NCORE, Np, F)

    ht2 = pl.pallas_call(
        _mid_body,
        grid=grid,
        in_specs=[
            deg_spec,
            half_spec,
            pl.BlockSpec((1, H), lambda i: (0, 0)),
            pl.BlockSpec((H, D), lambda i: (0, 0)),
        ],
        out_specs=half_spec,
        out_shape=jax.ShapeDtypeStruct((NCORE, Np, F), jnp.float32),
    )(deg_parts, sum1, b1.reshape(1, H), W2)

    sum2 = agg_kernel(ht2.reshape(NCORE * Np, F), srcall,
                      dst_e).reshape(NCORE, Np, F)

    hfin = pl.pallas_call(
        _fin_body,
        grid=grid,
        in_specs=[
            deg_spec,
            half_spec,
            pl.BlockSpec((1, D), lambda i: (0, 0)),
        ],
        out_specs=pl.BlockSpec((RB, D), lambda i: (i, 0)),
        out_shape=jax.ShapeDtypeStruct((Np, D), jnp.float32),
    )(deg_parts, sum2, b2.reshape(1, D))

    pair_kernel = _make_pair_kernel(Np, 2 * B, D)
    gcn_pair = pair_kernel(hfin, pidx).reshape(B, 2 * D)

    ns, ne, np_ = scg_pair.shape[1], esm_pair.shape[1], gpt_pair.shape[1]
    wg = Wc1[:2 * D]
    ws = Wc1[2 * D:2 * D + ns]
    we = Wc1[2 * D + ns:2 * D + ns + ne]
    wp = Wc1[2 * D + ns + ne:]
    w2p = jnp.zeros((128, 128), jnp.float32).at[:, :Wc2.shape[1]].set(Wc2)
    b2p = jnp.zeros((1, 128), jnp.float32).at[:, :Wc2.shape[1]].set(bc2[None, :])
    w3p = jnp.zeros((128, 128), jnp.float32).at[:Wc3.shape[0], :Wc3.shape[1]].set(Wc3)
    b3p = jnp.zeros((1, 128), jnp.float32).at[:, :Wc3.shape[1]].set(bc3[None, :])
    PB = 512
    full = lambda r, c: pl.BlockSpec((r, c), lambda i: (0, 0))
    out_pad = pl.pallas_call(
        _mlp_body,
        grid=(B // PB,),
        in_specs=[
            pl.BlockSpec((PB, 2 * D), lambda i: (i, 0)),
            pl.BlockSpec((PB, ns), lambda i: (i, 0)),
            pl.BlockSpec((PB, ne), lambda i: (i, 0)),
            pl.BlockSpec((PB, np_), lambda i: (i, 0)),
            full(2 * D, 128), full(ns, 128), full(ne, 128), full(np_, 128),
            full(1, 128), full(128, 128), full(1, 128), full(128, 128),
            full(1, 128),
        ],
        out_specs=pl.BlockSpec((PB, 128), lambda i: (i, 0)),
        out_shape=jax.ShapeDtypeStruct((B, 128), jnp.float32),
    )(gcn_pair, scg_pair, esm_pair, gpt_pair, wg, ws, we, wp,
      bc1.reshape(1, 128), w2p, b2p, w3p, b3p)
    return out_pad[:, :Wc3.shape[1]]
